# baseline (speedup 1.0000x reference)
import jax
import jax.numpy as jnp
from jax import lax
from jax.experimental import pallas as pl
from jax.experimental.pallas import tpu as pltpu
from jax.experimental.pallas import tpu_sc as plsc

WORD_LEN = 20
NUM_WORDS = 100000
CHAR_VOCAB = 1000
D = 64
N_NODES = 16384
L = 16
NQ = D // L
NC, NS = 2, 16
NW = NC * NS
NPT = N_NODES // NW
NG = NPT // L
IDX_CHUNK = 128
N_ELEM = NPT * WORD_LEN
N_CHUNKS = N_ELEM // IDX_CHUNK
DMA_BATCH = 16
N_BATCHES = N_CHUNKS // DMA_BATCH
TOK_WORDS = 100000 * WORD_LEN
TBL_WORDS = CHAR_VOCAB * D


def _sc_body(tokens_hbm, table_hbm, ids_hbm, out_hbm,
             ids_v, idxe_v, tok_v, table_v, out_v, sem, sem_t):
    wid = lax.axis_index("s") * NC + lax.axis_index("c")

    pltpu.sync_copy(ids_hbm.at[pl.ds(wid * NPT, NPT)], ids_v)
    table_cp = pltpu.async_copy(table_hbm, table_v, sem_t)

    def idx_body(g, carry):
        base = g * L
        idv = ids_v[pl.ds(base, L)]
        for c in range(WORD_LEN):
            idxe_v[pl.ds(c * NPT + base, L)] = idv + c * NUM_WORDS
        return carry

    lax.fori_loop(0, NG, idx_body, 0)

    def dma_body(o, carry):
        copies = [
            pltpu.async_copy(
                tokens_hbm.at[idxe_v.at[pl.ds(
                    (o * DMA_BATCH + b) * IDX_CHUNK, IDX_CHUNK)]],
                tok_v.at[pl.ds((o * DMA_BATCH + b) * IDX_CHUNK, IDX_CHUNK)],
                sem)
            for b in range(DMA_BATCH)
        ]
        for cp in copies:
            cp.wait()
        return carry

    lax.fori_loop(0, N_BATCHES, dma_body, 0)
    table_cp.wait()

    zeros = jnp.zeros((L,), jnp.float32)
    for q in range(NQ):
        table_v[pl.ds(q * L, L)] = zeros

    one = jnp.ones((L,), jnp.float32)
    zero = jnp.zeros((L,), jnp.float32)

    @plsc.parallel_loop(0, NG)
    def group_body(g):
        base = g * L
        toks = [tok_v[pl.ds(c * NPT + base, L)] for c in range(WORD_LEN)]
        cnt = zero
        for c in range(WORD_LEN):
            cnt = cnt + jnp.where(toks[c] != 0, one, zero)
        inv = one / jnp.maximum(cnt, one)
        for u in range(L):
            accs = [zero] * NQ
            accs2 = [zero] * NQ
            for c in range(WORD_LEN // 2):
                t64a = toks[2 * c][u] * D
                t64b = toks[2 * c + 1][u] * D
                for q in range(NQ):
                    accs[q] = accs[q] + table_v[pl.ds(t64a + q * L, L)]
                    accs2[q] = accs2[q] + table_v[pl.ds(t64b + q * L, L)]
            inv_u = inv[u]
            for q in range(NQ):
                out_v[pl.ds((base + u) * D + q * L, L)] = (
                    (accs[q] + accs2[q]) * inv_u)

    pltpu.sync_copy(out_v, out_hbm.at[pl.ds(wid * NPT * D, NPT * D)])


@jax.jit
def _run(tokens_flat, table_flat, ids):
    mesh = plsc.VectorSubcoreMesh(
        core_axis_name="c", subcore_axis_name="s",
        num_cores=NC, num_subcores=NS)
    f = pl.kernel(
        _sc_body,
        out_type=jax.ShapeDtypeStruct((N_NODES * D,), jnp.float32),
        mesh=mesh,
        compiler_params=pltpu.CompilerParams(needs_layout_passes=False),
        scratch_types=[
            pltpu.VMEM((NPT,), jnp.int32),
            pltpu.VMEM((N_ELEM,), jnp.int32),
            pltpu.VMEM((N_ELEM,), jnp.int32),
            pltpu.VMEM((TBL_WORDS,), jnp.float32),
            pltpu.VMEM((NPT * D,), jnp.float32),
            pltpu.SemaphoreType.DMA,
            pltpu.SemaphoreType.DMA,
        ],
    )
    return f(tokens_flat, table_flat, ids).reshape(N_NODES, D)


def kernel(local_char_embedding_tokens, lookup_ids, char_table, special_vectors):
    del special_vectors
    tokens_flat = local_char_embedding_tokens.astype(jnp.int32).T.reshape(-1)
    table_flat = char_table.reshape(CHAR_VOCAB * D)
    return _run(tokens_flat, table_flat, lookup_ids.astype(jnp.int32))


# speedup vs baseline: 1.0549x; 1.0549x over previous
import jax
import jax.numpy as jnp
from jax import lax
from jax.experimental import pallas as pl
from jax.experimental.pallas import tpu as pltpu
from jax.experimental.pallas import tpu_sc as plsc

WORD_LEN = 20
NUM_WORDS = 100000
CHAR_VOCAB = 1000
D = 64
N_NODES = 16384
L = 16
NQ = D // L
NC, NS = 2, 16
NW = NC * NS
NPT = N_NODES // NW
NG = NPT // L
IDX_CHUNK = 128
N_ELEM = NPT * WORD_LEN
N_CHUNKS = N_ELEM // IDX_CHUNK
DMA_BATCH = 16
N_BATCHES = N_CHUNKS // DMA_BATCH
NGH = NG // 2
NPH = NPT // 2
TOK_WORDS = 100000 * WORD_LEN
TBL_WORDS = CHAR_VOCAB * D


def _sc_body(tokens_hbm, table_hbm, ids_hbm, out_hbm,
             ids_v, idxe_v, tok_v, table_v, out_v, sem, sem_t):
    wid = lax.axis_index("s") * NC + lax.axis_index("c")

    pltpu.sync_copy(ids_hbm.at[pl.ds(wid * NPT, NPT)], ids_v)
    table_cp = pltpu.async_copy(table_hbm, table_v, sem_t)

    def idx_body(g, carry):
        base = g * L
        idv = ids_v[pl.ds(base, L)]
        for c in range(WORD_LEN):
            idxe_v[pl.ds(c * NPT + base, L)] = idv + c * NUM_WORDS
        return carry

    lax.fori_loop(0, NG, idx_body, 0)

    def dma_body(o, carry):
        copies = [
            pltpu.async_copy(
                tokens_hbm.at[idxe_v.at[pl.ds(
                    (o * DMA_BATCH + b) * IDX_CHUNK, IDX_CHUNK)]],
                tok_v.at[pl.ds((o * DMA_BATCH + b) * IDX_CHUNK, IDX_CHUNK)],
                sem)
            for b in range(DMA_BATCH)
        ]
        for cp in copies:
            cp.wait()
        return carry

    lax.fori_loop(0, N_BATCHES, dma_body, 0)
    table_cp.wait()

    zeros = jnp.zeros((L,), jnp.float32)
    for q in range(NQ):
        table_v[pl.ds(q * L, L)] = zeros

    one = jnp.ones((L,), jnp.float32)
    zero = jnp.zeros((L,), jnp.float32)

    def group_body(g, carry):
        @pl.when(g == NGH)
        def _():
            pltpu.sync_copy(out_v, out_hbm.at[pl.ds(wid * NPT, NPH)])

        base = g * L
        lrow = (g % NGH) * L
        toks = [tok_v[pl.ds(c * NPT + base, L)] for c in range(WORD_LEN)]
        cnt = zero
        for c in range(WORD_LEN):
            cnt = cnt + jnp.where(toks[c] != 0, one, zero)
        inv = one / jnp.maximum(cnt, one)
        for u in range(L):
            accs = [zero] * NQ
            for c in range(WORD_LEN):
                t64 = toks[c][u] * D
                for q in range(NQ):
                    accs[q] = accs[q] + table_v[pl.ds(t64 + q * L, L)]
            inv_u = inv[u]
            for q in range(NQ):
                out_v[lrow + u, pl.ds(q * L, L)] = accs[q] * inv_u
        return carry

    lax.fori_loop(0, NG, group_body, 0)

    pltpu.sync_copy(out_v, out_hbm.at[pl.ds(wid * NPT + NPH, NPH)])


@jax.jit
def _run(tokens_flat, table_flat, ids):
    mesh = plsc.VectorSubcoreMesh(
        core_axis_name="c", subcore_axis_name="s",
        num_cores=NC, num_subcores=NS)
    f = pl.kernel(
        _sc_body,
        out_type=jax.ShapeDtypeStruct((N_NODES, D), jnp.float32),
        mesh=mesh,
        compiler_params=pltpu.CompilerParams(needs_layout_passes=False),
        scratch_types=[
            pltpu.VMEM((NPT,), jnp.int32),
            pltpu.VMEM((N_ELEM,), jnp.int32),
            pltpu.VMEM((N_ELEM,), jnp.int32),
            pltpu.VMEM((TBL_WORDS,), jnp.float32),
            pltpu.VMEM((NPH, D), jnp.float32),
            pltpu.SemaphoreType.DMA,
            pltpu.SemaphoreType.DMA,
        ],
    )
    return f(tokens_flat, table_flat, ids)


def kernel(local_char_embedding_tokens, lookup_ids, char_table, special_vectors):
    del special_vectors
    tokens_flat = local_char_embedding_tokens.astype(jnp.int32).T.reshape(-1)
    table_flat = char_table.reshape(CHAR_VOCAB * D)
    return _run(tokens_flat, table_flat, lookup_ids.astype(jnp.int32))


# speedup vs baseline: 1.2909x; 1.2237x over previous
import jax
import jax.numpy as jnp
from jax import lax
from jax.experimental import pallas as pl
from jax.experimental.pallas import tpu as pltpu
from jax.experimental.pallas import tpu_sc as plsc

WORD_LEN = 20
NUM_WORDS = 100000
CHAR_VOCAB = 1000
D = 64
N_NODES = 16384
L = 16
NQ = D // L
NC, NS = 2, 16
NW = NC * NS
NPT = N_NODES // NW
NG = NPT // L
IDX_CHUNK = 128
N_ELEM = NPT * WORD_LEN
N_CHUNKS = N_ELEM // IDX_CHUNK
DMA_BATCH = 16
N_BATCHES = N_CHUNKS // DMA_BATCH
TOK_WORDS = 100000 * WORD_LEN
TBL_WORDS = CHAR_VOCAB * D


def _sc_body(tokens_hbm, table_hbm, ids_hbm, out_hbm,
             ids_v, idxe_v, tok_v, table_v, out_v, sem, sem_t):
    wid = lax.axis_index("s") * NC + lax.axis_index("c")

    pltpu.sync_copy(ids_hbm.at[pl.ds(wid * NPT, NPT)], ids_v)
    table_cp = pltpu.async_copy(table_hbm, table_v, sem_t)

    def idx_body(g, carry):
        base = g * L
        idv = ids_v[pl.ds(base, L)]
        for c in range(WORD_LEN):
            idxe_v[pl.ds(c * NPT + base, L)] = idv + c * NUM_WORDS
        return carry

    lax.fori_loop(0, NG, idx_body, 0)

    def dma_body(o, carry):
        copies = [
            pltpu.async_copy(
                tokens_hbm.at[idxe_v.at[pl.ds(
                    (o * DMA_BATCH + b) * IDX_CHUNK, IDX_CHUNK)]],
                tok_v.at[pl.ds((o * DMA_BATCH + b) * IDX_CHUNK, IDX_CHUNK)],
                sem)
            for b in range(DMA_BATCH)
        ]
        for cp in copies:
            cp.wait()
        return carry

    lax.fori_loop(0, N_BATCHES, dma_body, 0)
    table_cp.wait()

    zeros = jnp.zeros((L,), jnp.float32)
    for q in range(NQ):
        table_v[pl.ds(q * L, L)] = zeros

    one = jnp.ones((L,), jnp.float32)
    zero = jnp.zeros((L,), jnp.float32)

    @plsc.parallel_loop(0, NG)
    def group_body(g):
        base = g * L
        toks = [tok_v[pl.ds(c * NPT + base, L)] for c in range(WORD_LEN)]
        cnt = zero
        for c in range(WORD_LEN):
            cnt = cnt + jnp.where(toks[c] != 0, one, zero)
        inv = one / jnp.maximum(cnt, one)
        for u in range(L):
            accs = [zero] * NQ
            for c in range(WORD_LEN):
                t64 = toks[c][u] * D
                for q in range(NQ):
                    accs[q] = accs[q] + table_v[pl.ds(t64 + q * L, L)]
            inv_u = inv[u]
            for q in range(NQ):
                out_v[pl.ds((base + u) * D + q * L, L)] = accs[q] * inv_u

    pltpu.sync_copy(out_v, out_hbm.at[pl.ds(wid * NPT * D, NPT * D)])


@jax.jit
def _run(tokens_flat, table_flat, ids):
    mesh = plsc.VectorSubcoreMesh(
        core_axis_name="c", subcore_axis_name="s",
        num_cores=NC, num_subcores=NS)
    f = pl.kernel(
        _sc_body,
        out_type=jax.ShapeDtypeStruct((N_NODES * D,), jnp.float32),
        mesh=mesh,
        compiler_params=pltpu.CompilerParams(needs_layout_passes=False),
        scratch_types=[
            pltpu.VMEM((NPT,), jnp.int32),
            pltpu.VMEM((N_ELEM,), jnp.int32),
            pltpu.VMEM((N_ELEM,), jnp.int32),
            pltpu.VMEM((TBL_WORDS,), jnp.float32),
            pltpu.VMEM((NPT * D,), jnp.float32),
            pltpu.SemaphoreType.DMA,
            pltpu.SemaphoreType.DMA,
        ],
    )
    return f(tokens_flat, table_flat, ids).reshape(N_NODES, D)


def kernel(local_char_embedding_tokens, lookup_ids, char_table, special_vectors):
    del special_vectors
    tokens_flat = local_char_embedding_tokens.astype(jnp.int32).T.reshape(-1)
    table_flat = char_table.reshape(CHAR_VOCAB * D)
    return _run(tokens_flat, table_flat, lookup_ids.astype(jnp.int32))


# speedup vs baseline: 1.2976x; 1.0051x over previous
import jax
import jax.numpy as jnp
from jax import lax
from jax.experimental import pallas as pl
from jax.experimental.pallas import tpu as pltpu
from jax.experimental.pallas import tpu_sc as plsc

WORD_LEN = 20
NUM_WORDS = 100000
CHAR_VOCAB = 1000
D = 64
N_NODES = 16384
L = 16
NQ = D // L
NC, NS = 2, 16
NW = NC * NS
NPT = N_NODES // NW
NG = NPT // L
IDX_CHUNK = 128
N_ELEM = NPT * WORD_LEN
N_CHUNKS = N_ELEM // IDX_CHUNK
DMA_BATCH = 16
N_BATCHES = N_CHUNKS // DMA_BATCH
TBL_WORDS = CHAR_VOCAB * D


def _sc_body(tokens_hbm, table_hbm, ids_hbm, out_hbm,
             ids_v, idxe_v, tok_v, table_v, out_v, sem, sem_t):
    wid = lax.axis_index("s") * NC + lax.axis_index("c")

    pltpu.sync_copy(ids_hbm.at[pl.ds(wid * NPT, NPT)], ids_v)
    table_cp = pltpu.async_copy(table_hbm, table_v, sem_t)

    def idx_body(g, carry):
        base = g * L
        idv = ids_v[pl.ds(base, L)]
        for c in range(WORD_LEN):
            idxe_v[pl.ds(c * NPT + base, L)] = idv + c * NUM_WORDS
        return carry

    lax.fori_loop(0, NG, idx_body, 0)

    def dma_body(o, carry):
        copies = [
            pltpu.async_copy(
                tokens_hbm.at[idxe_v.at[pl.ds(
                    (o * DMA_BATCH + b) * IDX_CHUNK, IDX_CHUNK)]],
                tok_v.at[pl.ds((o * DMA_BATCH + b) * IDX_CHUNK, IDX_CHUNK)],
                sem)
            for b in range(DMA_BATCH)
        ]
        for cp in copies:
            cp.wait()
        return carry

    lax.fori_loop(0, N_BATCHES, dma_body, 0)
    table_cp.wait()

    zeros = jnp.zeros((L,), jnp.float32)
    for q in range(NQ):
        table_v[pl.ds(q * L, L)] = zeros

    one = jnp.ones((L,), jnp.float32)
    zero = jnp.zeros((L,), jnp.float32)

    @plsc.parallel_loop(0, NG)
    def group_body(g):
        base = g * L
        toks = [tok_v[pl.ds(c * NPT + base, L)] for c in range(WORD_LEN)]
        cnt = zero
        for c in range(WORD_LEN):
            cnt = cnt + jnp.where(toks[c] != 0, one, zero)
        inv = one / jnp.maximum(cnt, one)
        for u in range(L):
            accs = [zero] * NQ
            for c in range(WORD_LEN):
                t64 = toks[c][u] * D
                for q in range(NQ):
                    accs[q] = accs[q] + table_v[pl.ds(t64 + q * L, L)]
            inv_u = inv[u]
            for q in range(NQ):
                out_v[pl.ds((base + u) * D + q * L, L)] = accs[q] * inv_u

    pltpu.sync_copy(out_v, out_hbm.at[pl.ds(wid * NPT * D, NPT * D)])


@jax.jit
def _run(tokens_flat, table_flat, ids):
    mesh = plsc.VectorSubcoreMesh(
        core_axis_name="c", subcore_axis_name="s",
        num_cores=NC, num_subcores=NS)
    f = pl.kernel(
        _sc_body,
        out_type=jax.ShapeDtypeStruct((N_NODES * D,), jnp.float32),
        mesh=mesh,
        compiler_params=pltpu.CompilerParams(needs_layout_passes=False),
        scratch_types=[
            pltpu.VMEM((NPT,), jnp.int32),
            pltpu.VMEM((N_ELEM,), jnp.int32),
            pltpu.VMEM((N_ELEM,), jnp.int32),
            pltpu.VMEM((TBL_WORDS,), jnp.float32),
            pltpu.VMEM((NPT * D,), jnp.float32),
            pltpu.SemaphoreType.DMA,
            pltpu.SemaphoreType.DMA,
        ],
    )
    return f(tokens_flat, table_flat, ids).reshape(N_NODES, D)


def kernel(local_char_embedding_tokens, lookup_ids, char_table, special_vectors):
    del special_vectors
    tokens_flat = local_char_embedding_tokens.astype(jnp.int32).T.reshape(-1)
    table_flat = char_table.reshape(CHAR_VOCAB * D)
    return _run(tokens_flat, table_flat, lookup_ids.astype(jnp.int32))
